# baseline (speedup 1.0000x reference)
import functools

import jax
import jax.numpy as jnp
from jax import lax
from jax.experimental import pallas as pl
from jax.experimental.pallas import tpu as pltpu
from jax.experimental.pallas import tpu_sc as plsc

N = 4608
K = 8192
D = 256
BN = 512
BK = 512


def _argmin_body(x_ref, cb_ref, xsq_ref, csq_ref, idx_ref, bv_ref, bi_ref):
    j = pl.program_id(1)

    @pl.when(j == 0)
    def _():
        bv_ref[...] = jnp.full((BN, 1), jnp.inf, jnp.float32)
        bi_ref[...] = jnp.zeros((BN, 1), jnp.int32)

    x = x_ref[...]
    cb = cb_ref[...]
    xc = lax.dot_general(x, cb, (((1,), (1,)), ((), ())),
                         preferred_element_type=jnp.float32)
    d2 = (xsq_ref[...] - 2.0 * xc) + csq_ref[...]
    scores = jnp.sqrt(jnp.maximum(d2, 0.0))
    bmin = jnp.min(scores, axis=1, keepdims=True)
    ids = lax.broadcasted_iota(jnp.int32, scores.shape, 1) + j * BK
    bidx = jnp.min(jnp.where(scores <= bmin, ids, jnp.int32(K)),
                   axis=1, keepdims=True)
    upd = bmin < bv_ref[...]
    bi_ref[...] = jnp.where(upd, bidx, bi_ref[...])
    bv_ref[...] = jnp.where(upd, bmin, bv_ref[...])

    @pl.when(j == pl.num_programs(1) - 1)
    def _():
        idx_ref[...] = bi_ref[...]


def _nearest_indices(x, codebook):
    x_sq = jnp.sum(x * x, axis=-1, keepdims=True)
    c_sq = jnp.sum(codebook * codebook, axis=-1)[None, :]
    idx2 = pl.pallas_call(
        _argmin_body,
        grid=(N // BN, K // BK),
        in_specs=[
            pl.BlockSpec((BN, D), lambda i, j: (i, 0)),
            pl.BlockSpec((BK, D), lambda i, j: (j, 0)),
            pl.BlockSpec((BN, 1), lambda i, j: (i, 0)),
            pl.BlockSpec((1, BK), lambda i, j: (0, j)),
        ],
        out_specs=pl.BlockSpec((BN, 1), lambda i, j: (i, 0)),
        out_shape=jax.ShapeDtypeStruct((N, 1), jnp.int32),
        scratch_shapes=[
            pltpu.VMEM((BN, 1), jnp.float32),
            pltpu.VMEM((BN, 1), jnp.int32),
        ],
    )(x, codebook, x_sq, c_sq)
    return idx2.reshape(N)


def _make_sc_gather():
    info = plsc.get_sparse_core_info()
    nc, ns = info.num_cores, info.num_subcores
    nw = nc * ns
    bpw = N // nw
    mesh = plsc.VectorSubcoreMesh(core_axis_name="c", subcore_axis_name="s")

    @functools.partial(
        pl.kernel, mesh=mesh,
        out_type=jax.ShapeDtypeStruct((N, D), jnp.float32),
        scratch_types=[
            pltpu.VMEM((bpw,), jnp.int32),
            pltpu.VMEM((bpw, D), jnp.float32),
            pltpu.SemaphoreType.DMA,
        ],
    )
    def gather_k(table_hbm, idx_hbm, out_hbm, idx_v, rows_v, sem):
        wid = lax.axis_index("s") * nc + lax.axis_index("c")
        base = wid * bpw
        pltpu.sync_copy(idx_hbm.at[pl.ds(base, bpw)], idx_v)
        pltpu.async_copy(table_hbm.at[idx_v], rows_v, sem).wait()
        pltpu.sync_copy(rows_v, out_hbm.at[pl.ds(base, bpw)])

    return gather_k


def kernel(x, codebook, embed_table):
    indices = _nearest_indices(x, codebook)
    return _make_sc_gather()(embed_table, indices)


# speedup vs baseline: 1.2391x; 1.2391x over previous
import functools

import jax
import jax.numpy as jnp
from jax import lax
from jax.experimental import pallas as pl
from jax.experimental.pallas import tpu as pltpu
from jax.experimental.pallas import tpu_sc as plsc

N = 4608
K = 8192
D = 256
BN = 512
BK = 1024


def _argmin_body(x_ref, cb_ref, xsq_ref, csq_ref, idx_ref, bv_ref, bi_ref):
    j = pl.program_id(1)

    @pl.when(j == 0)
    def _():
        bv_ref[...] = jnp.full((BN, 1), jnp.inf, jnp.float32)
        bi_ref[...] = jnp.zeros((BN, 1), jnp.int32)

    x = x_ref[...]
    cb2 = cb_ref[...]
    xc2 = lax.dot_general(x, cb2, (((1,), (1,)), ((), ())),
                          preferred_element_type=jnp.float32)
    d2 = (xsq_ref[...] + xc2) + csq_ref[...]
    scores = jnp.sqrt(jnp.maximum(d2, 0.0))
    bmin = jnp.min(scores, axis=1, keepdims=True)
    ids = lax.broadcasted_iota(jnp.int32, scores.shape, 1) + j * BK
    bidx = jnp.min(jnp.where(scores <= bmin, ids, jnp.int32(K)),
                   axis=1, keepdims=True)
    upd = bmin < bv_ref[...]
    bi_ref[...] = jnp.where(upd, bidx, bi_ref[...])
    bv_ref[...] = jnp.where(upd, bmin, bv_ref[...])

    @pl.when(j == pl.num_programs(1) - 1)
    def _():
        idx_ref[...] = bi_ref[...]


def _nearest_indices(x, codebook):
    x_sq = jnp.sum(x * x, axis=-1, keepdims=True)
    c_sq = jnp.sum(codebook * codebook, axis=-1)[None, :]
    codebook = -2.0 * codebook
    idx2 = pl.pallas_call(
        _argmin_body,
        grid=(N // BN, K // BK),
        in_specs=[
            pl.BlockSpec((BN, D), lambda i, j: (i, 0)),
            pl.BlockSpec((BK, D), lambda i, j: (j, 0)),
            pl.BlockSpec((BN, 1), lambda i, j: (i, 0)),
            pl.BlockSpec((1, BK), lambda i, j: (0, j)),
        ],
        out_specs=pl.BlockSpec((BN, 1), lambda i, j: (i, 0)),
        out_shape=jax.ShapeDtypeStruct((N, 1), jnp.int32),
        scratch_shapes=[
            pltpu.VMEM((BN, 1), jnp.float32),
            pltpu.VMEM((BN, 1), jnp.int32),
        ],
    )(x, codebook, x_sq, c_sq)
    return idx2.reshape(N)


def _make_sc_gather():
    info = plsc.get_sparse_core_info()
    nc, ns = info.num_cores, info.num_subcores
    nw = nc * ns
    bpw = N // nw
    mesh = plsc.VectorSubcoreMesh(core_axis_name="c", subcore_axis_name="s")

    @functools.partial(
        pl.kernel, mesh=mesh,
        out_type=jax.ShapeDtypeStruct((N, D), jnp.float32),
        scratch_types=[
            pltpu.VMEM((bpw,), jnp.int32),
            pltpu.VMEM((bpw, D), jnp.float32),
            pltpu.SemaphoreType.DMA,
        ],
    )
    def gather_k(table_hbm, idx_hbm, out_hbm, idx_v, rows_v, sem):
        wid = lax.axis_index("s") * nc + lax.axis_index("c")
        base = wid * bpw
        pltpu.sync_copy(idx_hbm.at[pl.ds(base, bpw)], idx_v)
        pltpu.async_copy(table_hbm.at[idx_v], rows_v, sem).wait()
        pltpu.sync_copy(rows_v, out_hbm.at[pl.ds(base, bpw)])

    return gather_k


def kernel(x, codebook, embed_table):
    indices = _nearest_indices(x, codebook)
    return _make_sc_gather()(embed_table, indices)


# speedup vs baseline: 1.4151x; 1.1421x over previous
import functools

import jax
import jax.numpy as jnp
from jax import lax
from jax.experimental import pallas as pl
from jax.experimental.pallas import tpu as pltpu
from jax.experimental.pallas import tpu_sc as plsc

N = 4608
K = 8192
D = 256
BN = 512
BK = 1024


LANES = 128


def _argmin_body(x_ref, cb_ref, xsq_ref, csq_ref, idx_ref, vm_ref, vi_ref):
    j = pl.program_id(1)

    @pl.when(j == 0)
    def _():
        vm_ref[...] = jnp.full((BN, LANES), jnp.inf, jnp.float32)
        vi_ref[...] = jnp.zeros((BN, LANES), jnp.int32)

    x = x_ref[...]
    cb2 = cb_ref[...]
    xc2 = lax.dot_general(x, cb2, (((1,), (1,)), ((), ())),
                          preferred_element_type=jnp.float32)
    d2 = (xsq_ref[...] + xc2) + csq_ref[...]
    scores = jnp.sqrt(jnp.maximum(d2, 0.0))

    lane_iota = lax.broadcasted_iota(jnp.int32, (BN, LANES), 1)
    vm = vm_ref[...]
    vi = vi_ref[...]
    for g in range(BK // LANES):
        s = scores[:, g * LANES:(g + 1) * LANES]
        kid = lane_iota + (j * BK + g * LANES)
        m = s < vm
        vm = jnp.where(m, s, vm)
        vi = jnp.where(m, kid, vi)
    vm_ref[...] = vm
    vi_ref[...] = vi

    @pl.when(j == pl.num_programs(1) - 1)
    def _():
        gm = jnp.min(vm, axis=1, keepdims=True)
        cand = jnp.where(vm <= gm, vi, jnp.int32(K))
        idx_ref[...] = jnp.min(cand, axis=1, keepdims=True)


def _nearest_indices(x, codebook):
    x_sq = jnp.sum(x * x, axis=-1, keepdims=True)
    c_sq = jnp.sum(codebook * codebook, axis=-1)[None, :]
    codebook = -2.0 * codebook
    idx2 = pl.pallas_call(
        _argmin_body,
        grid=(N // BN, K // BK),
        in_specs=[
            pl.BlockSpec((BN, D), lambda i, j: (i, 0)),
            pl.BlockSpec((BK, D), lambda i, j: (j, 0)),
            pl.BlockSpec((BN, 1), lambda i, j: (i, 0)),
            pl.BlockSpec((1, BK), lambda i, j: (0, j)),
        ],
        out_specs=pl.BlockSpec((BN, 1), lambda i, j: (i, 0)),
        out_shape=jax.ShapeDtypeStruct((N, 1), jnp.int32),
        scratch_shapes=[
            pltpu.VMEM((BN, LANES), jnp.float32),
            pltpu.VMEM((BN, LANES), jnp.int32),
        ],
    )(x, codebook, x_sq, c_sq)
    return idx2.reshape(N)


def _make_sc_gather():
    info = plsc.get_sparse_core_info()
    nc, ns = info.num_cores, info.num_subcores
    nw = nc * ns
    bpw = N // nw
    mesh = plsc.VectorSubcoreMesh(core_axis_name="c", subcore_axis_name="s")

    @functools.partial(
        pl.kernel, mesh=mesh,
        out_type=jax.ShapeDtypeStruct((N, D), jnp.float32),
        scratch_types=[
            pltpu.VMEM((bpw,), jnp.int32),
            pltpu.VMEM((bpw, D), jnp.float32),
            pltpu.SemaphoreType.DMA,
        ],
    )
    def gather_k(table_hbm, idx_hbm, out_hbm, idx_v, rows_v, sem):
        wid = lax.axis_index("s") * nc + lax.axis_index("c")
        base = wid * bpw
        pltpu.sync_copy(idx_hbm.at[pl.ds(base, bpw)], idx_v)
        pltpu.async_copy(table_hbm.at[idx_v], rows_v, sem).wait()
        pltpu.sync_copy(rows_v, out_hbm.at[pl.ds(base, bpw)])

    return gather_k


def kernel(x, codebook, embed_table):
    indices = _nearest_indices(x, codebook)
    return _make_sc_gather()(embed_table, indices)
